# baseline (speedup 1.0000x reference)
import functools

import jax
import jax.numpy as jnp
from jax.experimental import pallas as pl
from jax.experimental.pallas import tpu as pltpu

_T = 128
_TB = 512


def _segment_matmul_kernel(ti_ref, ei_ref, sr_ref, er_ref,
                           hs_ref, cw0_ref, cw1_ref, cw2_ref, cw3_ref,
                           cb_ref, out_ref):
    T = hs_ref.shape[0]
    g = pl.program_id(0)
    start = sr_ref[g]
    end = er_ref[g]

    @pl.when(end > start)
    def _():
        x = hs_ref[...].astype(jnp.bfloat16)
        K = cw0_ref.shape[1]
        logits = cb_ref[0].astype(jnp.float32)
        for k, cw_ref in enumerate((cw0_ref, cw1_ref, cw2_ref, cw3_ref)):
            xk = x[:, k * K:(k + 1) * K]
            wk = cw_ref[0].astype(jnp.bfloat16)
            logits = logits + jnp.dot(xk, wk,
                                      preferred_element_type=jnp.float32)
        r_iota = jax.lax.broadcasted_iota(jnp.int32, (T, 1), 0)
        active = (r_iota >= start) & (r_iota < end)
        out_ref[...] = jnp.where(active, logits, out_ref[...])


def _lse_rows(logits):
    m = jnp.max(logits, axis=1, keepdims=True)
    return jnp.log(jnp.sum(jnp.exp(logits - m), axis=1, keepdims=True)) + m


def _epilogue_kernel(cl_ref, hs_ref, wr_ref, br_ref, cols_ref, rows_ref,
                     out_ref):
    T = cl_ref.shape[0]
    E = cl_ref.shape[1]
    b = pl.program_id(0)

    @pl.when(b == 0)
    def _():
        out_ref[0] = jnp.float32(0.0)

    c_iota = jax.lax.broadcasted_iota(jnp.int32, (T, E), 1)
    cl = cl_ref[...]
    c_lse = _lse_rows(cl)
    c_tgt = jnp.sum(jnp.where(c_iota == cols_ref[:, 0:1], cl, 0.0),
                    axis=1, keepdims=True)
    x = hs_ref[...].astype(jnp.bfloat16)
    rl = jnp.dot(x, wr_ref[...].astype(jnp.bfloat16),
                 preferred_element_type=jnp.float32) + br_ref[...]
    r_lse = _lse_rows(rl)
    r_tgt = jnp.sum(jnp.where(c_iota == rows_ref[:, 0:1], rl, 0.0),
                    axis=1, keepdims=True)
    out_ref[0] += jnp.sum((c_lse - c_tgt) + (r_lse - r_tgt))


def kernel(hidden_states, target_ids, W_row, b_row, col_weight, col_bias):
    E, D = W_row.shape
    Bb, S, _ = hidden_states.shape
    N = Bb * S
    T = _T
    num_tiles = N // T
    G = num_tiles + E

    ids = target_ids.reshape(-1).astype(jnp.int32)
    row_ids = ids // E
    col_ids = ids % E

    sort_idx = jnp.argsort(row_ids).astype(jnp.int32)
    row_sorted = jnp.take(row_ids, sort_idx)
    col_sorted = jnp.take(col_ids, sort_idx)
    hs_flat = hidden_states.reshape(N, D)
    hs_sorted = jnp.take(hs_flat, sort_idx, axis=0)

    counts = jnp.zeros((E,), jnp.int32).at[row_ids].add(1)
    offsets = (jnp.cumsum(counts) - counts).astype(jnp.int32)
    tile_starts = jnp.arange(num_tiles, dtype=jnp.int32) * T
    seg_starts = jnp.sort(jnp.concatenate([tile_starts, offsets]))
    seg_ends = jnp.concatenate([seg_starts[1:], jnp.array([N], jnp.int32)])
    tile_of = jnp.minimum(seg_starts // T, num_tiles - 1)
    expert_of = jnp.take(row_sorted, jnp.minimum(seg_starts, N - 1))
    start_rel = seg_starts - tile_of * T
    end_rel = seg_ends - tile_of * T

    cb3 = col_bias.reshape(E, 1, E)

    K = D // 4

    def _cw_spec(k):
        return pl.BlockSpec((1, K, E), lambda g, ti, ei, sr, er, _k=k: (ei[g], _k, 0))

    seg_spec = pltpu.PrefetchScalarGridSpec(
        num_scalar_prefetch=4,
        grid=(G,),
        in_specs=[
            pl.BlockSpec((T, D), lambda g, ti, ei, sr, er: (ti[g], 0)),
            _cw_spec(0), _cw_spec(1), _cw_spec(2), _cw_spec(3),
            pl.BlockSpec((1, 1, E), lambda g, ti, ei, sr, er: (ei[g], 0, 0)),
        ],
        out_specs=pl.BlockSpec((T, E), lambda g, ti, ei, sr, er: (ti[g], 0)),
    )
    col_logits = pl.pallas_call(
        _segment_matmul_kernel,
        grid_spec=seg_spec,
        out_shape=jax.ShapeDtypeStruct((N, E), jnp.float32),
    )(tile_of, expert_of, start_rel, end_rel, hs_sorted,
      col_weight, col_weight, col_weight, col_weight, cb3)

    cols_bcast = jnp.broadcast_to(col_sorted[:, None], (N, 128))
    rows_bcast = jnp.broadcast_to(row_sorted[:, None], (N, 128))
    w_row_t = W_row.T
    b_row2 = b_row.reshape(1, E)

    TB = _TB
    nb = N // TB
    total = pl.pallas_call(
        _epilogue_kernel,
        grid=(nb,),
        in_specs=[
            pl.BlockSpec((TB, E), lambda b: (b, 0)),
            pl.BlockSpec((TB, D), lambda b: (b, 0)),
            pl.BlockSpec((D, E), lambda b: (0, 0)),
            pl.BlockSpec((1, E), lambda b: (0, 0)),
            pl.BlockSpec((TB, 128), lambda b: (b, 0)),
            pl.BlockSpec((TB, 128), lambda b: (b, 0)),
        ],
        out_specs=pl.BlockSpec((1,), lambda b: (0,), memory_space=pltpu.SMEM),
        out_shape=jax.ShapeDtypeStruct((1,), jnp.float32),
    )(col_logits, hs_sorted, w_row_t, b_row2, cols_bcast, rows_bcast)

    return total[0] / jnp.float32(N)


# speedup vs baseline: 1.0586x; 1.0586x over previous
import functools

import jax
import jax.numpy as jnp
from jax.experimental import pallas as pl
from jax.experimental.pallas import tpu as pltpu

_T = 256
_TB = 512


def _segment_matmul_kernel(ti_ref, ei_ref, sr_ref, er_ref,
                           hs_ref, cw0_ref, cw1_ref, cw2_ref, cw3_ref,
                           cb_ref, out_ref):
    T = hs_ref.shape[0]
    g = pl.program_id(0)
    start = sr_ref[g]
    end = er_ref[g]

    @pl.when(end > start)
    def _():
        x = hs_ref[...].astype(jnp.bfloat16)
        K = cw0_ref.shape[1]
        logits = cb_ref[0].astype(jnp.float32)
        for k, cw_ref in enumerate((cw0_ref, cw1_ref, cw2_ref, cw3_ref)):
            xk = x[:, k * K:(k + 1) * K]
            wk = cw_ref[0].astype(jnp.bfloat16)
            logits = logits + jnp.dot(xk, wk,
                                      preferred_element_type=jnp.float32)
        r_iota = jax.lax.broadcasted_iota(jnp.int32, (T, 1), 0)
        active = (r_iota >= start) & (r_iota < end)
        out_ref[...] = jnp.where(active, logits, out_ref[...])


def _lse_rows(logits):
    m = jnp.max(logits, axis=1, keepdims=True)
    return jnp.log(jnp.sum(jnp.exp(logits - m), axis=1, keepdims=True)) + m


def _epilogue_kernel(cl_ref, hs_ref, wr_ref, br_ref, cols_ref, rows_ref,
                     out_ref):
    T = cl_ref.shape[0]
    E = cl_ref.shape[1]
    b = pl.program_id(0)

    @pl.when(b == 0)
    def _():
        out_ref[0] = jnp.float32(0.0)

    c_iota = jax.lax.broadcasted_iota(jnp.int32, (T, E), 1)
    cl = cl_ref[...]
    c_lse = _lse_rows(cl)
    c_tgt = jnp.sum(jnp.where(c_iota == cols_ref[:, 0:1], cl, 0.0),
                    axis=1, keepdims=True)
    x = hs_ref[...].astype(jnp.bfloat16)
    rl = jnp.dot(x, wr_ref[...].astype(jnp.bfloat16),
                 preferred_element_type=jnp.float32) + br_ref[...]
    r_lse = _lse_rows(rl)
    r_tgt = jnp.sum(jnp.where(c_iota == rows_ref[:, 0:1], rl, 0.0),
                    axis=1, keepdims=True)
    out_ref[0] += jnp.sum((c_lse - c_tgt) + (r_lse - r_tgt))


def kernel(hidden_states, target_ids, W_row, b_row, col_weight, col_bias):
    E, D = W_row.shape
    Bb, S, _ = hidden_states.shape
    N = Bb * S
    T = _T
    num_tiles = N // T
    G = num_tiles + E

    ids = target_ids.reshape(-1).astype(jnp.int32)
    row_ids = ids // E
    col_ids = ids % E

    sort_idx = jnp.argsort(row_ids).astype(jnp.int32)
    row_sorted = jnp.take(row_ids, sort_idx)
    col_sorted = jnp.take(col_ids, sort_idx)
    hs_flat = hidden_states.reshape(N, D)
    hs_sorted = jnp.take(hs_flat, sort_idx, axis=0)

    counts = jnp.zeros((E,), jnp.int32).at[row_ids].add(1)
    offsets = (jnp.cumsum(counts) - counts).astype(jnp.int32)
    tile_starts = jnp.arange(num_tiles, dtype=jnp.int32) * T
    seg_starts = jnp.sort(jnp.concatenate([tile_starts, offsets]))
    seg_ends = jnp.concatenate([seg_starts[1:], jnp.array([N], jnp.int32)])
    tile_of = jnp.minimum(seg_starts // T, num_tiles - 1)
    expert_of = jnp.take(row_sorted, jnp.minimum(seg_starts, N - 1))
    start_rel = seg_starts - tile_of * T
    end_rel = seg_ends - tile_of * T

    cb3 = col_bias.reshape(E, 1, E)

    K = D // 4

    def _cw_spec(k):
        return pl.BlockSpec((1, K, E), lambda g, ti, ei, sr, er, _k=k: (ei[g], _k, 0))

    seg_spec = pltpu.PrefetchScalarGridSpec(
        num_scalar_prefetch=4,
        grid=(G,),
        in_specs=[
            pl.BlockSpec((T, D), lambda g, ti, ei, sr, er: (ti[g], 0)),
            _cw_spec(0), _cw_spec(1), _cw_spec(2), _cw_spec(3),
            pl.BlockSpec((1, 1, E), lambda g, ti, ei, sr, er: (ei[g], 0, 0)),
        ],
        out_specs=pl.BlockSpec((T, E), lambda g, ti, ei, sr, er: (ti[g], 0)),
    )
    col_logits = pl.pallas_call(
        _segment_matmul_kernel,
        grid_spec=seg_spec,
        out_shape=jax.ShapeDtypeStruct((N, E), jnp.float32),
    )(tile_of, expert_of, start_rel, end_rel, hs_sorted,
      col_weight, col_weight, col_weight, col_weight, cb3)

    cols_bcast = jnp.broadcast_to(col_sorted[:, None], (N, 128))
    rows_bcast = jnp.broadcast_to(row_sorted[:, None], (N, 128))
    w_row_t = W_row.T
    b_row2 = b_row.reshape(1, E)

    TB = _TB
    nb = N // TB
    total = pl.pallas_call(
        _epilogue_kernel,
        grid=(nb,),
        in_specs=[
            pl.BlockSpec((TB, E), lambda b: (b, 0)),
            pl.BlockSpec((TB, D), lambda b: (b, 0)),
            pl.BlockSpec((D, E), lambda b: (0, 0)),
            pl.BlockSpec((1, E), lambda b: (0, 0)),
            pl.BlockSpec((TB, 128), lambda b: (b, 0)),
            pl.BlockSpec((TB, 128), lambda b: (b, 0)),
        ],
        out_specs=pl.BlockSpec((1,), lambda b: (0,), memory_space=pltpu.SMEM),
        out_shape=jax.ShapeDtypeStruct((1,), jnp.float32),
    )(col_logits, hs_sorted, w_row_t, b_row2, cols_bcast, rows_bcast)

    return total[0] / jnp.float32(N)


# speedup vs baseline: 1.0663x; 1.0073x over previous
import functools

import jax
import jax.numpy as jnp
from jax.experimental import pallas as pl
from jax.experimental.pallas import tpu as pltpu

_T = 256
_TB = 512


def _segment_matmul_kernel(ti_ref, ei_ref, sr_ref, er_ref,
                           hs_ref, cw0_ref, cw1_ref, cw2_ref, cw3_ref,
                           cb_ref, out_ref):
    T = hs_ref.shape[0]
    g = pl.program_id(0)
    start = sr_ref[g]
    end = er_ref[g]

    @pl.when(end > start)
    def _():
        x = hs_ref[...].astype(jnp.bfloat16)
        K = cw0_ref.shape[1]
        logits = cb_ref[0].astype(jnp.float32)
        for k, cw_ref in enumerate((cw0_ref, cw1_ref, cw2_ref, cw3_ref)):
            xk = x[:, k * K:(k + 1) * K]
            wk = cw_ref[0].astype(jnp.bfloat16)
            logits = logits + jnp.dot(xk, wk,
                                      preferred_element_type=jnp.float32)
        r_iota = jax.lax.broadcasted_iota(jnp.int32, (T, 1), 0)
        active = (r_iota >= start) & (r_iota < end)
        out_ref[...] = jnp.where(active, logits, out_ref[...])


def _lse_rows(logits):
    m = jnp.max(logits, axis=1, keepdims=True)
    return jnp.log(jnp.sum(jnp.exp(logits - m), axis=1, keepdims=True)) + m


def _col_epilogue_kernel(cl_ref, cols_ref, out_ref):
    T = cl_ref.shape[0]
    E = cl_ref.shape[1]
    b = pl.program_id(0)

    @pl.when(b == 0)
    def _():
        out_ref[0] = jnp.float32(0.0)

    c_iota = jax.lax.broadcasted_iota(jnp.int32, (T, E), 1)
    cl = cl_ref[...]
    c_lse = _lse_rows(cl)
    c_tgt = jnp.sum(jnp.where(c_iota == cols_ref[:, 0:1], cl, 0.0),
                    axis=1, keepdims=True)
    out_ref[0] += jnp.sum(c_lse - c_tgt)


def _row_loss_kernel(hs_ref, wr_ref, br_ref, rows_ref, out_ref):
    T = hs_ref.shape[0]
    E = br_ref.shape[1]
    b = pl.program_id(0)

    @pl.when(b == 0)
    def _():
        out_ref[0] = jnp.float32(0.0)

    c_iota = jax.lax.broadcasted_iota(jnp.int32, (T, E), 1)
    x = hs_ref[...].astype(jnp.bfloat16)
    rl = jnp.dot(x, wr_ref[...].astype(jnp.bfloat16),
                 preferred_element_type=jnp.float32) + br_ref[...]
    r_lse = _lse_rows(rl)
    r_tgt = jnp.sum(jnp.where(c_iota == rows_ref[:, 0:1], rl, 0.0),
                    axis=1, keepdims=True)
    out_ref[0] += jnp.sum(r_lse - r_tgt)


def kernel(hidden_states, target_ids, W_row, b_row, col_weight, col_bias):
    E, D = W_row.shape
    Bb, S, _ = hidden_states.shape
    N = Bb * S
    T = _T
    num_tiles = N // T
    G = num_tiles + E

    ids = target_ids.reshape(-1).astype(jnp.int32)
    row_ids = ids // E
    col_ids = ids % E

    sort_idx = jnp.argsort(row_ids).astype(jnp.int32)
    row_sorted = jnp.take(row_ids, sort_idx)
    col_sorted = jnp.take(col_ids, sort_idx)
    hs_flat = hidden_states.reshape(N, D)
    hs_sorted = jnp.take(hs_flat, sort_idx, axis=0)

    counts = jnp.zeros((E,), jnp.int32).at[row_ids].add(1)
    offsets = (jnp.cumsum(counts) - counts).astype(jnp.int32)
    tile_starts = jnp.arange(num_tiles, dtype=jnp.int32) * T
    seg_starts = jnp.sort(jnp.concatenate([tile_starts, offsets]))
    seg_ends = jnp.concatenate([seg_starts[1:], jnp.array([N], jnp.int32)])
    tile_of = jnp.minimum(seg_starts // T, num_tiles - 1)
    expert_of = jnp.take(row_sorted, jnp.minimum(seg_starts, N - 1))
    start_rel = seg_starts - tile_of * T
    end_rel = seg_ends - tile_of * T

    cb3 = col_bias.reshape(E, 1, E)

    K = D // 4

    def _cw_spec(k):
        return pl.BlockSpec((1, K, E), lambda g, ti, ei, sr, er, _k=k: (ei[g], _k, 0))

    seg_spec = pltpu.PrefetchScalarGridSpec(
        num_scalar_prefetch=4,
        grid=(G,),
        in_specs=[
            pl.BlockSpec((T, D), lambda g, ti, ei, sr, er: (ti[g], 0)),
            _cw_spec(0), _cw_spec(1), _cw_spec(2), _cw_spec(3),
            pl.BlockSpec((1, 1, E), lambda g, ti, ei, sr, er: (ei[g], 0, 0)),
        ],
        out_specs=pl.BlockSpec((T, E), lambda g, ti, ei, sr, er: (ti[g], 0)),
    )
    col_logits = pl.pallas_call(
        _segment_matmul_kernel,
        grid_spec=seg_spec,
        out_shape=jax.ShapeDtypeStruct((N, E), jnp.float32),
    )(tile_of, expert_of, start_rel, end_rel, hs_sorted,
      col_weight, col_weight, col_weight, col_weight, cb3)

    cols_bcast = jnp.broadcast_to(col_sorted[:, None], (N, 128))
    rows_u_bcast = jnp.broadcast_to(row_ids[:, None], (N, 128))
    w_row_t = W_row.T
    b_row2 = b_row.reshape(1, E)

    TB = _TB
    nb = N // TB
    row_total = pl.pallas_call(
        _row_loss_kernel,
        grid=(nb,),
        in_specs=[
            pl.BlockSpec((TB, D), lambda b: (b, 0)),
            pl.BlockSpec((D, E), lambda b: (0, 0)),
            pl.BlockSpec((1, E), lambda b: (0, 0)),
            pl.BlockSpec((TB, 128), lambda b: (b, 0)),
        ],
        out_specs=pl.BlockSpec((1,), lambda b: (0,), memory_space=pltpu.SMEM),
        out_shape=jax.ShapeDtypeStruct((1,), jnp.float32),
    )(hs_flat, w_row_t, b_row2, rows_u_bcast)

    col_total = pl.pallas_call(
        _col_epilogue_kernel,
        grid=(nb,),
        in_specs=[
            pl.BlockSpec((TB, E), lambda b: (b, 0)),
            pl.BlockSpec((TB, 128), lambda b: (b, 0)),
        ],
        out_specs=pl.BlockSpec((1,), lambda b: (0,), memory_space=pltpu.SMEM),
        out_shape=jax.ShapeDtypeStruct((1,), jnp.float32),
    )(col_logits, cols_bcast)

    return (row_total[0] + col_total[0]) / jnp.float32(N)


# speedup vs baseline: 1.1234x; 1.0535x over previous
import functools

import jax
import jax.numpy as jnp
from jax.experimental import pallas as pl
from jax.experimental.pallas import tpu as pltpu

_T = 256
_TB = 1024


def _segment_matmul_kernel(ti_ref, ei_ref, sr_ref, er_ref,
                           hs_ref, cw_ref, cb_ref, out_ref):
    T = hs_ref.shape[0]
    g = pl.program_id(0)
    start = sr_ref[g]
    end = er_ref[g]

    @pl.when(end > start)
    def _():
        x = hs_ref[...].astype(jnp.bfloat16)
        w = cw_ref[0].astype(jnp.bfloat16)
        logits = jnp.dot(x, w, preferred_element_type=jnp.float32) + cb_ref[0]
        r_iota = jax.lax.broadcasted_iota(jnp.int32, (T, 1), 0)
        active = (r_iota >= start) & (r_iota < end)
        out_ref[...] = jnp.where(active, logits, out_ref[...])


def _lse_rows(logits):
    m = jnp.max(logits, axis=1, keepdims=True)
    return jnp.log(jnp.sum(jnp.exp(logits - m), axis=1, keepdims=True)) + m


def _col_epilogue_kernel(cl_ref, cols_ref, out_ref):
    T = cl_ref.shape[0]
    E = cl_ref.shape[1]
    b = pl.program_id(0)

    @pl.when(b == 0)
    def _():
        out_ref[0] = jnp.float32(0.0)

    c_iota = jax.lax.broadcasted_iota(jnp.int32, (T, E), 1)
    cl = cl_ref[...]
    c_lse = _lse_rows(cl)
    c_tgt = jnp.sum(jnp.where(c_iota == cols_ref[:, 0:1], cl, 0.0),
                    axis=1, keepdims=True)
    out_ref[0] += jnp.sum(c_lse - c_tgt)


def _row_loss_kernel(hs_ref, wr_ref, br_ref, rows_ref, out_ref):
    T = hs_ref.shape[0]
    E = br_ref.shape[1]
    b = pl.program_id(0)

    @pl.when(b == 0)
    def _():
        out_ref[0] = jnp.float32(0.0)

    c_iota = jax.lax.broadcasted_iota(jnp.int32, (T, E), 1)
    x = hs_ref[...].astype(jnp.bfloat16)
    rl = jnp.dot(x, wr_ref[...].astype(jnp.bfloat16),
                 preferred_element_type=jnp.float32) + br_ref[...]
    r_lse = _lse_rows(rl)
    r_tgt = jnp.sum(jnp.where(c_iota == rows_ref[:, 0:1], rl, 0.0),
                    axis=1, keepdims=True)
    out_ref[0] += jnp.sum(r_lse - r_tgt)


def kernel(hidden_states, target_ids, W_row, b_row, col_weight, col_bias):
    E, D = W_row.shape
    Bb, S, _ = hidden_states.shape
    N = Bb * S
    T = _T
    num_tiles = N // T
    G = num_tiles + E

    ids = target_ids.reshape(-1).astype(jnp.int32)
    row_ids = ids // E
    col_ids = ids % E

    sort_idx = jnp.argsort(row_ids).astype(jnp.int32)
    row_sorted = jnp.take(row_ids, sort_idx)
    col_sorted = jnp.take(col_ids, sort_idx)
    hs_flat = hidden_states.reshape(N, D)
    hs_sorted = jnp.take(hs_flat, sort_idx, axis=0)

    counts = jnp.zeros((E,), jnp.int32).at[row_ids].add(1)
    offsets = (jnp.cumsum(counts) - counts).astype(jnp.int32)
    tile_starts = jnp.arange(num_tiles, dtype=jnp.int32) * T
    seg_starts = jnp.sort(jnp.concatenate([tile_starts, offsets]))
    seg_ends = jnp.concatenate([seg_starts[1:], jnp.array([N], jnp.int32)])
    tile_of = jnp.minimum(seg_starts // T, num_tiles - 1)
    expert_of = jnp.take(row_sorted, jnp.minimum(seg_starts, N - 1))
    start_rel = seg_starts - tile_of * T
    end_rel = seg_ends - tile_of * T

    cb3 = col_bias.reshape(E, 1, E)

    seg_spec = pltpu.PrefetchScalarGridSpec(
        num_scalar_prefetch=4,
        grid=(G,),
        in_specs=[
            pl.BlockSpec((T, D), lambda g, ti, ei, sr, er: (ti[g], 0)),
            pl.BlockSpec((1, D, E), lambda g, ti, ei, sr, er: (ei[g], 0, 0)),
            pl.BlockSpec((1, 1, E), lambda g, ti, ei, sr, er: (ei[g], 0, 0)),
        ],
        out_specs=pl.BlockSpec((T, E), lambda g, ti, ei, sr, er: (ti[g], 0)),
    )
    col_logits = pl.pallas_call(
        _segment_matmul_kernel,
        grid_spec=seg_spec,
        out_shape=jax.ShapeDtypeStruct((N, E), jnp.float32),
    )(tile_of, expert_of, start_rel, end_rel, hs_sorted, col_weight, cb3)

    cols_bcast = jnp.broadcast_to(col_sorted[:, None], (N, 128))
    rows_u_bcast = jnp.broadcast_to(row_ids[:, None], (N, 128))
    w_row_t = W_row.T
    b_row2 = b_row.reshape(1, E)

    TB = _TB
    nb = N // TB
    row_total = pl.pallas_call(
        _row_loss_kernel,
        grid=(nb,),
        in_specs=[
            pl.BlockSpec((TB, D), lambda b: (b, 0)),
            pl.BlockSpec((D, E), lambda b: (0, 0)),
            pl.BlockSpec((1, E), lambda b: (0, 0)),
            pl.BlockSpec((TB, 128), lambda b: (b, 0)),
        ],
        out_specs=pl.BlockSpec((1,), lambda b: (0,), memory_space=pltpu.SMEM),
        out_shape=jax.ShapeDtypeStruct((1,), jnp.float32),
    )(hs_flat, w_row_t, b_row2, rows_u_bcast)

    col_total = pl.pallas_call(
        _col_epilogue_kernel,
        grid=(nb,),
        in_specs=[
            pl.BlockSpec((TB, E), lambda b: (b, 0)),
            pl.BlockSpec((TB, 128), lambda b: (b, 0)),
        ],
        out_specs=pl.BlockSpec((1,), lambda b: (0,), memory_space=pltpu.SMEM),
        out_shape=jax.ShapeDtypeStruct((1,), jnp.float32),
    )(col_logits, cols_bcast)

    return (row_total[0] + col_total[0]) / jnp.float32(N)
